# baseline (speedup 1.0000x reference)
import functools

import jax
import jax.numpy as jnp
from jax import lax
from jax.experimental import pallas as pl
from jax.experimental.pallas import tpu as pltpu
from jax.experimental.pallas import tpu_sc as plsc

_NC, _NS = 2, 16
_NW = _NC * _NS
_PN = 320
_NPAD = _NW * _PN
_F = 128
_CH = 4032
_GH = 128
_G = 3 * _GH
_FLUSH_AT = _G - 32
_LPAD = 2048


_FW = _F // 2


def _zero_acc(acc):
    def zero_body(i, _):
        for j in range(_FW // 16):
            acc[i, pl.ds(16 * j, 16)] = jnp.zeros((16,), jnp.int32)
        return 0
    lax.fori_loop(0, _PN + 8, zero_body, 0)


def _merge_accs(acc0, acc1):
    def merge_body(i, _):
        for j in range(_FW // 16):
            a = plsc.bitcast(acc0[i, pl.ds(16 * j, 16)], jnp.bfloat16)
            b = plsc.bitcast(acc1[i, pl.ds(16 * j, 16)], jnp.bfloat16)
            acc0[i, pl.ds(16 * j, 16)] = plsc.bitcast(jnp.maximum(a, b), jnp.int32)
        return 0
    lax.fori_loop(0, _PN, merge_body, 0)


def _gather_and_accumulate(hp, idxb, dlb, rows, acc0, acc1, semg):
    cs = [pltpu.async_copy(hp.at[idxb.at[pl.ds(k * _GH, _GH)]], rows.at[k], semg)
          for k in range(_G // _GH)]
    for c in cs:
        c.wait()
    _accumulate_rows(dlb, rows, acc0, acc1)


def _accumulate_rows(dlb, rows, acc0, acc1):
    accs = (acc0, acc1)
    for b in range(_G // _GH):
        def grp_body(g, _, b=b):
            dvec = dlb[pl.ds(b * _GH + g * 16, 16)]
            for l in range(16):
                d = dvec[l]
                e = g * 16 + l
                acc = accs[l & 1]
                for j in range(_FW // 16):
                    a = plsc.bitcast(acc[d, pl.ds(16 * j, 16)], jnp.bfloat16)
                    r = plsc.bitcast(rows[b, e, pl.ds(16 * j, 16)], jnp.bfloat16)
                    acc[d, pl.ds(16 * j, 16)] = plsc.bitcast(
                        jnp.maximum(a, r), jnp.int32)
            return 0
        lax.fori_loop(0, _GH // 16, grp_body, 0)


def _segmax_scan_body(sb, hp, srcp, dstp, out, sidx, sdl, counts,
                      acc, acc1, srcb0, srcb1, dstb0, dstb1, idxb, dlb, rows,
                      stgi, stgd, cntv, semld, semg, semst):
    srcb = (srcb0, srcb1)
    dstb = (dstb0, dstb1)
    wid = lax.axis_index("s") * _NC + lax.axis_index("c")
    lo = wid * _PN
    tb = wid * (sb * _G)
    nchunks = srcp.shape[0] // _CH

    _zero_acc(acc)
    _zero_acc(acc1)
    for g in range(_G // 16 + 1):
        idxb[pl.ds(g * 16, 16)] = jnp.zeros((16,), jnp.int32)
        dlb[pl.ds(g * 16, 16)] = jnp.full((16,), _PN, jnp.int32)

    def flush(nb, nblk):
        del nb
        @pl.when(nblk > 0)
        def _():
            pltpu.make_async_copy(srcp.at[pl.ds(0, _G)], stgi, semst).wait()
            pltpu.make_async_copy(srcp.at[pl.ds(0, _G)], stgd, semst).wait()
        for g in range(_G // 16):
            stgi[pl.ds(g * 16, 16)] = idxb[pl.ds(g * 16, 16)]
            stgd[pl.ds(g * 16, 16)] = dlb[pl.ds(g * 16, 16)]
        off = tb + nblk * _G
        pltpu.async_copy(stgi, sidx.at[pl.ds(off, _G)], semst)
        pltpu.async_copy(stgd, sdl.at[pl.ds(off, _G)], semst)
        _gather_and_accumulate(hp, idxb, dlb, rows, acc, acc1, semg)

    def issue_chunk(ci, b):
        off = ci * _CH
        pltpu.async_copy(srcp.at[pl.ds(off, _CH)], srcb[b], semld.at[b])
        pltpu.async_copy(dstp.at[pl.ds(off, _CH)], dstb[b], semld.at[b])

    def wait_chunk(b):
        pltpu.make_async_copy(srcp.at[pl.ds(0, _CH)], srcb[b], semld.at[b]).wait()
        pltpu.make_async_copy(dstp.at[pl.ds(0, _CH)], dstb[b], semld.at[b]).wait()

    issue_chunk(0, 0)

    def outer_body(k, carry):
        for b in range(2):
            ci = 2 * k + b

            @pl.when(ci + 1 < nchunks)
            def _():
                issue_chunk(ci + 1, b ^ 1)
            wait_chunk(b)

            def pair_body(gp, carry):
                nb, nblk = carry
                for u in range(2):
                    base = gp * 32 + u * 16
                    dv = dstb[b][pl.ds(base, 16)]
                    sv = srcb[b][pl.ds(base, 16)]
                    dl = dv - lo
                    own = (dl >= 0) & (dl < _PN)
                    keys = jnp.where(own, dl, jnp.int32(_PN))
                    ks, vs = plsc.sort_key_val(keys, sv)
                    dlb[pl.ds(nb, 16)] = ks
                    idxb[pl.ds(nb, 16)] = vs
                    nb = nb + plsc.all_reduce_population_count(own)[0]

                def do_flush(args):
                    n, nk = args
                    flush(n, nk)
                    return jnp.int32(0), nk + 1
                return lax.cond(nb >= _FLUSH_AT, do_flush, lambda a: a, (nb, nblk))

            carry = lax.fori_loop(0, _CH // 32, pair_body, carry)
        return carry

    nb, nblk = lax.fori_loop(0, nchunks // 2, outer_body,
                             (jnp.int32(0), jnp.int32(0)))
    flush(nb, nblk)
    cntv[pl.ds(0, 16)] = jnp.full((16,), nblk + 1, jnp.int32)
    pltpu.sync_copy(cntv, counts.at[pl.ds(wid * 16, 16)])
    pltpu.make_async_copy(srcp.at[pl.ds(0, _G)], stgi, semst).wait()
    pltpu.make_async_copy(srcp.at[pl.ds(0, _G)], stgd, semst).wait()
    _merge_accs(acc, acc1)
    pltpu.sync_copy(acc.at[pl.ds(0, _PN)], out.at[pl.ds(lo, _PN)])


def _segmax_replay_body(sb, hp, sidx, sdl, counts, out,
                        acc, acc1, idxb0, idxb1, dlb0, dlb1, rows0, rows1,
                        cntv, semg):
    idxb = (idxb0, idxb1)
    dlb = (dlb0, dlb1)
    rows = (rows0, rows1)
    wid = lax.axis_index("s") * _NC + lax.axis_index("c")
    lo = wid * _PN
    tb = wid * (sb * _G)

    _zero_acc(acc)
    _zero_acc(acc1)
    pltpu.sync_copy(counts.at[pl.ds(wid * 16, 16)], cntv)
    cnt = cntv[pl.ds(0, 16)][0]

    def fetch(bi, p):
        off = tb + bi * _G
        pltpu.sync_copy(sidx.at[pl.ds(off, _G + 16)], idxb[p])
        pltpu.sync_copy(sdl.at[pl.ds(off, _G + 16)], dlb[p])
        for k in range(_G // _GH):
            pltpu.async_copy(hp.at[idxb[p].at[pl.ds(k * _GH, _GH)]],
                             rows[p].at[k], semg.at[p])

    def wait_rows(p):
        for k in range(_G // _GH):
            pltpu.make_async_copy(hp.at[pl.ds(0, _GH)], rows[p].at[k],
                                  semg.at[p]).wait()

    fetch(0, 0)

    def outer(i, _):
        for p in range(2):
            bi = 2 * i + p

            @pl.when(bi < cnt)
            def _():
                @pl.when(bi + 1 < cnt)
                def _():
                    fetch(bi + 1, p ^ 1)
                wait_rows(p)
                _accumulate_rows(dlb[p], rows[p], acc, acc1)
        return 0

    lax.fori_loop(0, (cnt + 1) // 2, outer, 0)
    _merge_accs(acc, acc1)
    pltpu.sync_copy(acc.at[pl.ds(0, _PN)], out.at[pl.ds(lo, _PN)])


def _sc_mesh():
    return plsc.VectorSubcoreMesh(core_axis_name="c", subcore_axis_name="s",
                                  num_cores=_NC, num_subcores=_NS)


def _segmax_scan(hp, src, dst):
    sb = src.shape[0] // _FLUSH_AT + 2
    sz = _NW * sb * _G + 16
    return pl.kernel(
        functools.partial(_segmax_scan_body, sb),
        out_type=[
            jax.ShapeDtypeStruct((_NPAD, _FW), jnp.int32),
            jax.ShapeDtypeStruct((sz,), jnp.int32),
            jax.ShapeDtypeStruct((sz,), jnp.int32),
            jax.ShapeDtypeStruct((_NW * 16,), jnp.int32),
        ],
        mesh=_sc_mesh(),
        scratch_types=[
            pltpu.VMEM((_PN + 8, _FW), jnp.int32),
            pltpu.VMEM((_PN + 8, _FW), jnp.int32),
            pltpu.VMEM((_CH,), jnp.int32),
            pltpu.VMEM((_CH,), jnp.int32),
            pltpu.VMEM((_CH,), jnp.int32),
            pltpu.VMEM((_CH,), jnp.int32),
            pltpu.VMEM((_G + 16,), jnp.int32),
            pltpu.VMEM((_G + 16,), jnp.int32),
            pltpu.VMEM((_G // _GH, _GH, _FW), jnp.int32),
            pltpu.VMEM((_G,), jnp.int32),
            pltpu.VMEM((_G,), jnp.int32),
            pltpu.VMEM((16,), jnp.int32),
            pltpu.SemaphoreType.DMA((2,)),
            pltpu.SemaphoreType.DMA,
            pltpu.SemaphoreType.DMA,
        ],
        compiler_params=pltpu.CompilerParams(needs_layout_passes=False, use_tc_tiling_on_sc=False),
        name="sc_segmax",
    )(hp, src, dst)


def _segmax_replay(hp, sidx, sdl, counts):
    sb = (sidx.shape[0] - 16) // (_NW * _G)
    return pl.kernel(
        functools.partial(_segmax_replay_body, sb),
        out_type=jax.ShapeDtypeStruct((_NPAD, _FW), jnp.int32),
        mesh=_sc_mesh(),
        scratch_types=[
            pltpu.VMEM((_PN + 8, _FW), jnp.int32),
            pltpu.VMEM((_PN + 8, _FW), jnp.int32),
            pltpu.VMEM((_G + 16,), jnp.int32),
            pltpu.VMEM((_G + 16,), jnp.int32),
            pltpu.VMEM((_G + 16,), jnp.int32),
            pltpu.VMEM((_G + 16,), jnp.int32),
            pltpu.VMEM((_G // _GH, _GH, _FW), jnp.int32),
            pltpu.VMEM((_G // _GH, _GH, _FW), jnp.int32),
            pltpu.VMEM((16,), jnp.int32),
            pltpu.SemaphoreType.DMA((2,)),
        ],
        compiler_params=pltpu.CompilerParams(needs_layout_passes=False, use_tc_tiling_on_sc=False),
        name="sc_segmax_replay",
    )(hp, sidx, sdl, counts)


def _leafgather_body(h, idx, out, idxv, rowsv, sem):
    wid = lax.axis_index("s") * _NC + lax.axis_index("c")
    per = _LPAD // _NW
    base = wid * per
    pltpu.sync_copy(idx.at[pl.ds(base, per)], idxv)
    pltpu.async_copy(h.at[idxv], rowsv, sem).wait()
    pltpu.sync_copy(rowsv, out.at[pl.ds(base, per)])


def _leafgather(h, idx):
    mesh = plsc.VectorSubcoreMesh(core_axis_name="c", subcore_axis_name="s",
                                  num_cores=_NC, num_subcores=_NS)
    per = _LPAD // _NW
    return pl.kernel(
        _leafgather_body,
        out_type=jax.ShapeDtypeStruct((_LPAD, _F), jnp.float32),
        mesh=mesh,
        scratch_types=[
            pltpu.VMEM((per,), jnp.int32),
            pltpu.VMEM((per, _F), jnp.float32),
            pltpu.SemaphoreType.DMA,
        ],
        name="sc_leafgather",
    )(h, idx)


def _pre_body(x_ref, Wp_ref, bp_ref, Ws_ref, hp_ref, s_ref):
    x = x_ref[...]
    hp_ref[...] = jnp.maximum(
        jnp.dot(x, Wp_ref[...], preferred_element_type=jnp.float32) + bp_ref[...],
        0.0).astype(jnp.bfloat16)
    s_ref[...] = jnp.dot(x, Ws_ref[...], preferred_element_type=jnp.float32)


def _pre(x, Wp, bp, Ws):
    grid = 8
    blk = _NPAD // grid
    return pl.pallas_call(
        _pre_body,
        grid=(grid,),
        in_specs=[
            pl.BlockSpec((blk, _F), lambda i: (i, 0)),
            pl.BlockSpec((_F, _F), lambda i: (0, 0)),
            pl.BlockSpec((1, _F), lambda i: (0, 0)),
            pl.BlockSpec((_F, _F), lambda i: (0, 0)),
        ],
        out_specs=[
            pl.BlockSpec((blk, _F), lambda i: (i, 0)),
            pl.BlockSpec((blk, _F), lambda i: (i, 0)),
        ],
        out_shape=[
            jax.ShapeDtypeStruct((_NPAD, _F), jnp.bfloat16),
            jax.ShapeDtypeStruct((_NPAD, _F), jnp.float32),
        ],
    )(x, Wp, bp.reshape(1, -1), Ws)


def _post_body(s_ref, ng_ref, Wn_ref, b_ref, h_ref):
    ng = ng_ref[...].astype(jnp.float32)
    h_ref[...] = jax.nn.sigmoid(
        s_ref[...]
        + jnp.dot(ng, Wn_ref[...], preferred_element_type=jnp.float32)
        + b_ref[...])


def _post(s, ng, Wn, b):
    grid = 8
    blk = _NPAD // grid
    return pl.pallas_call(
        _post_body,
        grid=(grid,),
        in_specs=[
            pl.BlockSpec((blk, _F), lambda i: (i, 0)),
            pl.BlockSpec((blk, _F), lambda i: (i, 0)),
            pl.BlockSpec((_F, _F), lambda i: (0, 0)),
            pl.BlockSpec((1, _F), lambda i: (0, 0)),
        ],
        out_specs=pl.BlockSpec((blk, _F), lambda i: (i, 0)),
        out_shape=jax.ShapeDtypeStruct((_NPAD, _F), jnp.float32),
    )(s, ng, Wn, b.reshape(1, -1))


def _mlp_body(le_ref, cmd_ref, Wc_ref, bc_ref, W3_ref, b3_ref, W4_ref, b4_ref,
              W5_ref, b5_ref, W6_ref, b6_ref, out_ref):
    enc = jnp.dot(cmd_ref[...], Wc_ref[...], preferred_element_type=jnp.float32) + bc_ref[...]
    prod = le_ref[...] * enc
    o = jax.nn.sigmoid(jnp.dot(prod, W3_ref[...], preferred_element_type=jnp.float32) + b3_ref[...])
    o = jax.nn.sigmoid(jnp.dot(o, W4_ref[...], preferred_element_type=jnp.float32) + b4_ref[...])
    o = jax.nn.sigmoid(jnp.dot(o, W5_ref[...], preferred_element_type=jnp.float32) + b5_ref[...])
    out_ref[...] = jax.nn.sigmoid(jnp.dot(o, W6_ref[...], preferred_element_type=jnp.float32) + b6_ref[...])


def _mlp(le, cmd, Wc, bc, W3, b3, W4, b4, W5, b5, W6, b6):
    return pl.pallas_call(
        _mlp_body,
        out_shape=jax.ShapeDtypeStruct((_LPAD, 1), jnp.float32),
    )(le, cmd.reshape(1, -1), Wc, bc.reshape(1, -1), W3, b3.reshape(1, -1),
      W4, b4.reshape(1, -1), W5, b5.reshape(1, -1), W6, b6.reshape(1, -1))


def kernel(node_inputs, edge_index, leaves, command, Wp1, bp1, Ws1, Wn1, b1,
           Wp2, bp2, Ws2, Wn2, b2, Wc, bc, W3, b3, W4, b4, W5, b5, W6, b6):
    N = node_inputs.shape[0]
    E = edge_index.shape[1]
    L = leaves.shape[0]

    x = jnp.pad(node_inputs, ((0, _NPAD - N), (0, 0)))
    src = edge_index[0]
    dst = edge_index[1]
    ep = -(-E // _CH) * _CH
    if (ep // _CH) % 2:
        ep += _CH
    if ep != E:
        src = jnp.pad(src, (0, ep - E))
        dst = jnp.pad(dst, (0, ep - E), constant_values=-1)

    def _pack(a):
        return lax.bitcast_convert_type(a.reshape(_NPAD, _FW, 2), jnp.int32)

    def _unpack(a):
        return lax.bitcast_convert_type(a, jnp.bfloat16).reshape(_NPAD, _F)

    hp1, s1 = _pre(x, Wp1, bp1, Ws1)
    ng1, sidx, sdl, counts = _segmax_scan(_pack(hp1), src, dst)
    h1 = _post(s1, _unpack(ng1), Wn1, b1)
    hp2, s2 = _pre(h1, Wp2, bp2, Ws2)
    ng2 = _segmax_replay(_pack(hp2), sidx, sdl, counts)
    h = _post(s2, _unpack(ng2), Wn2, b2)

    lv = jnp.pad(leaves, (0, _LPAD - L))
    le = _leafgather(h, lv)
    out = _mlp(le, command, Wc, bc, W3, b3, W4, b4, W5, b5, W6, b6)
    return out[:L]


# speedup vs baseline: 1.0270x; 1.0270x over previous
import functools

import jax
import jax.numpy as jnp
from jax import lax
from jax.experimental import pallas as pl
from jax.experimental.pallas import tpu as pltpu
from jax.experimental.pallas import tpu_sc as plsc

_NC, _NS = 2, 16
_NW = _NC * _NS
_PN = 320
_NPAD = _NW * _PN
_F = 128
_CH = 4032
_GH = 128
_G = 3 * _GH
_FLUSH_AT = _G - 32
_LPAD = 2048


_FW = _F // 2


def _zero_acc(acc):
    def zero_body(i, _):
        for j in range(_FW // 16):
            acc[i, pl.ds(16 * j, 16)] = jnp.zeros((16,), jnp.int32)
        return 0
    lax.fori_loop(0, _PN + 8, zero_body, 0)


def _merge_accs(acc0, acc1):
    def merge_body(i, _):
        for j in range(_FW // 16):
            a = plsc.bitcast(acc0[i, pl.ds(16 * j, 16)], jnp.bfloat16)
            b = plsc.bitcast(acc1[i, pl.ds(16 * j, 16)], jnp.bfloat16)
            acc0[i, pl.ds(16 * j, 16)] = plsc.bitcast(jnp.maximum(a, b), jnp.int32)
        return 0
    lax.fori_loop(0, _PN, merge_body, 0)


def _gather_and_accumulate(hp, idxb, dlb, rows, acc0, acc1, semg):
    cs = [pltpu.async_copy(hp.at[idxb.at[pl.ds(k * _GH, _GH)]], rows.at[k], semg)
          for k in range(_G // _GH)]
    for c in cs:
        c.wait()
    _accumulate_rows(dlb, rows, acc0, acc1)


def _accumulate_rows(dlb, rows, acc0, acc1):
    accs = (acc0, acc1)
    for b in range(_G // _GH):
        def grp_body(g, _, b=b):
            dvec = dlb[pl.ds(b * _GH + g * 16, 16)]
            for l in range(16):
                d = dvec[l]
                e = g * 16 + l
                acc = accs[l & 1]
                for j in range(_FW // 16):
                    a = plsc.bitcast(acc[d, pl.ds(16 * j, 16)], jnp.bfloat16)
                    r = plsc.bitcast(rows[b, e, pl.ds(16 * j, 16)], jnp.bfloat16)
                    acc[d, pl.ds(16 * j, 16)] = plsc.bitcast(
                        jnp.maximum(a, r), jnp.int32)
            return 0
        lax.fori_loop(0, _GH // 16, grp_body, 0)


def _segmax_scan_body(sb, hp, srcp, dstp, out, sidx, sdl, counts,
                      acc, acc1, srcb0, srcb1, dstb0, dstb1, idxb, dlb, rows,
                      stgi, stgd, cntv, semld, semg, semst):
    srcb = (srcb0, srcb1)
    dstb = (dstb0, dstb1)
    wid = lax.axis_index("s") * _NC + lax.axis_index("c")
    lo = wid * _PN
    tb = wid * (sb * _G)
    nchunks = srcp.shape[0] // _CH

    _zero_acc(acc)
    _zero_acc(acc1)
    for g in range(_G // 16 + 1):
        idxb[pl.ds(g * 16, 16)] = jnp.zeros((16,), jnp.int32)
        dlb[pl.ds(g * 16, 16)] = jnp.full((16,), _PN, jnp.int32)

    def flush(nb, nblk):
        del nb
        @pl.when(nblk > 0)
        def _():
            pltpu.make_async_copy(srcp.at[pl.ds(0, _G)], stgi, semst).wait()
            pltpu.make_async_copy(srcp.at[pl.ds(0, _G)], stgd, semst).wait()
        for g in range(_G // 16):
            stgi[pl.ds(g * 16, 16)] = idxb[pl.ds(g * 16, 16)]
            stgd[pl.ds(g * 16, 16)] = dlb[pl.ds(g * 16, 16)]
        off = tb + nblk * _G
        pltpu.async_copy(stgi, sidx.at[pl.ds(off, _G)], semst)
        pltpu.async_copy(stgd, sdl.at[pl.ds(off, _G)], semst)
        _gather_and_accumulate(hp, idxb, dlb, rows, acc, acc1, semg)

    def issue_chunk(ci, b):
        off = ci * _CH
        pltpu.async_copy(srcp.at[pl.ds(off, _CH)], srcb[b], semld.at[b])
        pltpu.async_copy(dstp.at[pl.ds(off, _CH)], dstb[b], semld.at[b])

    def wait_chunk(b):
        pltpu.make_async_copy(srcp.at[pl.ds(0, _CH)], srcb[b], semld.at[b]).wait()
        pltpu.make_async_copy(dstp.at[pl.ds(0, _CH)], dstb[b], semld.at[b]).wait()

    issue_chunk(0, 0)

    def outer_body(k, carry):
        for b in range(2):
            ci = 2 * k + b

            @pl.when(ci + 1 < nchunks)
            def _():
                issue_chunk(ci + 1, b ^ 1)
            wait_chunk(b)

            def pair_body(gp, carry):
                nb, nblk = carry
                for u in range(2):
                    base = gp * 32 + u * 16
                    dv = dstb[b][pl.ds(base, 16)]
                    sv = srcb[b][pl.ds(base, 16)]
                    dl = dv - lo
                    own = (dl >= 0) & (dl < _PN)
                    keys = jnp.where(own, dl, jnp.int32(_PN))
                    ks, vs = plsc.sort_key_val(keys, sv)
                    dlb[pl.ds(nb, 16)] = ks
                    idxb[pl.ds(nb, 16)] = vs
                    nb = nb + plsc.all_reduce_population_count(own)[0]

                def do_flush(args):
                    n, nk = args
                    flush(n, nk)
                    return jnp.int32(0), nk + 1
                return lax.cond(nb >= _FLUSH_AT, do_flush, lambda a: a, (nb, nblk))

            carry = lax.fori_loop(0, _CH // 32, pair_body, carry, unroll=2)
        return carry

    nb, nblk = lax.fori_loop(0, nchunks // 2, outer_body,
                             (jnp.int32(0), jnp.int32(0)))
    flush(nb, nblk)
    cntv[pl.ds(0, 16)] = jnp.full((16,), nblk + 1, jnp.int32)
    pltpu.sync_copy(cntv, counts.at[pl.ds(wid * 16, 16)])
    pltpu.make_async_copy(srcp.at[pl.ds(0, _G)], stgi, semst).wait()
    pltpu.make_async_copy(srcp.at[pl.ds(0, _G)], stgd, semst).wait()
    _merge_accs(acc, acc1)
    pltpu.sync_copy(acc.at[pl.ds(0, _PN)], out.at[pl.ds(lo, _PN)])


def _segmax_replay_body(sb, hp, sidx, sdl, counts, out,
                        acc, acc1, idxb0, idxb1, dlb0, dlb1, rows0, rows1,
                        cntv, semg):
    idxb = (idxb0, idxb1)
    dlb = (dlb0, dlb1)
    rows = (rows0, rows1)
    wid = lax.axis_index("s") * _NC + lax.axis_index("c")
    lo = wid * _PN
    tb = wid * (sb * _G)

    _zero_acc(acc)
    _zero_acc(acc1)
    pltpu.sync_copy(counts.at[pl.ds(wid * 16, 16)], cntv)
    cnt = cntv[pl.ds(0, 16)][0]

    def fetch(bi, p):
        off = tb + bi * _G
        pltpu.sync_copy(sidx.at[pl.ds(off, _G + 16)], idxb[p])
        pltpu.sync_copy(sdl.at[pl.ds(off, _G + 16)], dlb[p])
        for k in range(_G // _GH):
            pltpu.async_copy(hp.at[idxb[p].at[pl.ds(k * _GH, _GH)]],
                             rows[p].at[k], semg.at[p])

    def wait_rows(p):
        for k in range(_G // _GH):
            pltpu.make_async_copy(hp.at[pl.ds(0, _GH)], rows[p].at[k],
                                  semg.at[p]).wait()

    fetch(0, 0)

    def outer(i, _):
        for p in range(2):
            bi = 2 * i + p

            @pl.when(bi < cnt)
            def _():
                @pl.when(bi + 1 < cnt)
                def _():
                    fetch(bi + 1, p ^ 1)
                wait_rows(p)
                _accumulate_rows(dlb[p], rows[p], acc, acc1)
        return 0

    lax.fori_loop(0, (cnt + 1) // 2, outer, 0)
    _merge_accs(acc, acc1)
    pltpu.sync_copy(acc.at[pl.ds(0, _PN)], out.at[pl.ds(lo, _PN)])


def _sc_mesh():
    return plsc.VectorSubcoreMesh(core_axis_name="c", subcore_axis_name="s",
                                  num_cores=_NC, num_subcores=_NS)


def _segmax_scan(hp, src, dst):
    sb = src.shape[0] // _FLUSH_AT + 2
    sz = _NW * sb * _G + 16
    return pl.kernel(
        functools.partial(_segmax_scan_body, sb),
        out_type=[
            jax.ShapeDtypeStruct((_NPAD, _FW), jnp.int32),
            jax.ShapeDtypeStruct((sz,), jnp.int32),
            jax.ShapeDtypeStruct((sz,), jnp.int32),
            jax.ShapeDtypeStruct((_NW * 16,), jnp.int32),
        ],
        mesh=_sc_mesh(),
        scratch_types=[
            pltpu.VMEM((_PN + 8, _FW), jnp.int32),
            pltpu.VMEM((_PN + 8, _FW), jnp.int32),
            pltpu.VMEM((_CH,), jnp.int32),
            pltpu.VMEM((_CH,), jnp.int32),
            pltpu.VMEM((_CH,), jnp.int32),
            pltpu.VMEM((_CH,), jnp.int32),
            pltpu.VMEM((_G + 16,), jnp.int32),
            pltpu.VMEM((_G + 16,), jnp.int32),
            pltpu.VMEM((_G // _GH, _GH, _FW), jnp.int32),
            pltpu.VMEM((_G,), jnp.int32),
            pltpu.VMEM((_G,), jnp.int32),
            pltpu.VMEM((16,), jnp.int32),
            pltpu.SemaphoreType.DMA((2,)),
            pltpu.SemaphoreType.DMA,
            pltpu.SemaphoreType.DMA,
        ],
        compiler_params=pltpu.CompilerParams(needs_layout_passes=False, use_tc_tiling_on_sc=False),
        name="sc_segmax",
    )(hp, src, dst)


def _segmax_replay(hp, sidx, sdl, counts):
    sb = (sidx.shape[0] - 16) // (_NW * _G)
    return pl.kernel(
        functools.partial(_segmax_replay_body, sb),
        out_type=jax.ShapeDtypeStruct((_NPAD, _FW), jnp.int32),
        mesh=_sc_mesh(),
        scratch_types=[
            pltpu.VMEM((_PN + 8, _FW), jnp.int32),
            pltpu.VMEM((_PN + 8, _FW), jnp.int32),
            pltpu.VMEM((_G + 16,), jnp.int32),
            pltpu.VMEM((_G + 16,), jnp.int32),
            pltpu.VMEM((_G + 16,), jnp.int32),
            pltpu.VMEM((_G + 16,), jnp.int32),
            pltpu.VMEM((_G // _GH, _GH, _FW), jnp.int32),
            pltpu.VMEM((_G // _GH, _GH, _FW), jnp.int32),
            pltpu.VMEM((16,), jnp.int32),
            pltpu.SemaphoreType.DMA((2,)),
        ],
        compiler_params=pltpu.CompilerParams(needs_layout_passes=False, use_tc_tiling_on_sc=False),
        name="sc_segmax_replay",
    )(hp, sidx, sdl, counts)


def _leafgather_body(h, idx, out, idxv, rowsv, sem):
    wid = lax.axis_index("s") * _NC + lax.axis_index("c")
    per = _LPAD // _NW
    base = wid * per
    pltpu.sync_copy(idx.at[pl.ds(base, per)], idxv)
    pltpu.async_copy(h.at[idxv], rowsv, sem).wait()
    pltpu.sync_copy(rowsv, out.at[pl.ds(base, per)])


def _leafgather(h, idx):
    mesh = plsc.VectorSubcoreMesh(core_axis_name="c", subcore_axis_name="s",
                                  num_cores=_NC, num_subcores=_NS)
    per = _LPAD // _NW
    return pl.kernel(
        _leafgather_body,
        out_type=jax.ShapeDtypeStruct((_LPAD, _F), jnp.float32),
        mesh=mesh,
        scratch_types=[
            pltpu.VMEM((per,), jnp.int32),
            pltpu.VMEM((per, _F), jnp.float32),
            pltpu.SemaphoreType.DMA,
        ],
        name="sc_leafgather",
    )(h, idx)


def _pre_body(x_ref, Wp_ref, bp_ref, Ws_ref, hp_ref, s_ref):
    x = x_ref[...]
    hp_ref[...] = jnp.maximum(
        jnp.dot(x, Wp_ref[...], preferred_element_type=jnp.float32) + bp_ref[...],
        0.0).astype(jnp.bfloat16)
    s_ref[...] = jnp.dot(x, Ws_ref[...], preferred_element_type=jnp.float32)


def _pre(x, Wp, bp, Ws):
    grid = 8
    blk = _NPAD // grid
    return pl.pallas_call(
        _pre_body,
        grid=(grid,),
        in_specs=[
            pl.BlockSpec((blk, _F), lambda i: (i, 0)),
            pl.BlockSpec((_F, _F), lambda i: (0, 0)),
            pl.BlockSpec((1, _F), lambda i: (0, 0)),
            pl.BlockSpec((_F, _F), lambda i: (0, 0)),
        ],
        out_specs=[
            pl.BlockSpec((blk, _F), lambda i: (i, 0)),
            pl.BlockSpec((blk, _F), lambda i: (i, 0)),
        ],
        out_shape=[
            jax.ShapeDtypeStruct((_NPAD, _F), jnp.bfloat16),
            jax.ShapeDtypeStruct((_NPAD, _F), jnp.float32),
        ],
    )(x, Wp, bp.reshape(1, -1), Ws)


def _post_body(s_ref, ng_ref, Wn_ref, b_ref, h_ref):
    ng = ng_ref[...].astype(jnp.float32)
    h_ref[...] = jax.nn.sigmoid(
        s_ref[...]
        + jnp.dot(ng, Wn_ref[...], preferred_element_type=jnp.float32)
        + b_ref[...])


def _post(s, ng, Wn, b):
    grid = 8
    blk = _NPAD // grid
    return pl.pallas_call(
        _post_body,
        grid=(grid,),
        in_specs=[
            pl.BlockSpec((blk, _F), lambda i: (i, 0)),
            pl.BlockSpec((blk, _F), lambda i: (i, 0)),
            pl.BlockSpec((_F, _F), lambda i: (0, 0)),
            pl.BlockSpec((1, _F), lambda i: (0, 0)),
        ],
        out_specs=pl.BlockSpec((blk, _F), lambda i: (i, 0)),
        out_shape=jax.ShapeDtypeStruct((_NPAD, _F), jnp.float32),
    )(s, ng, Wn, b.reshape(1, -1))


def _post_pre_body(s_ref, ng_ref, Wn_ref, b_ref, Wp_ref, bp_ref, Ws_ref,
                   hp_ref, s2_ref):
    ng = ng_ref[...].astype(jnp.float32)
    h = jax.nn.sigmoid(
        s_ref[...]
        + jnp.dot(ng, Wn_ref[...], preferred_element_type=jnp.float32)
        + b_ref[...])
    hp_ref[...] = jnp.maximum(
        jnp.dot(h, Wp_ref[...], preferred_element_type=jnp.float32) + bp_ref[...],
        0.0).astype(jnp.bfloat16)
    s2_ref[...] = jnp.dot(h, Ws_ref[...], preferred_element_type=jnp.float32)


def _post_pre(s, ng, Wn, b, Wp, bp, Ws):
    grid = 8
    blk = _NPAD // grid
    return pl.pallas_call(
        _post_pre_body,
        grid=(grid,),
        in_specs=[
            pl.BlockSpec((blk, _F), lambda i: (i, 0)),
            pl.BlockSpec((blk, _F), lambda i: (i, 0)),
            pl.BlockSpec((_F, _F), lambda i: (0, 0)),
            pl.BlockSpec((1, _F), lambda i: (0, 0)),
            pl.BlockSpec((_F, _F), lambda i: (0, 0)),
            pl.BlockSpec((1, _F), lambda i: (0, 0)),
            pl.BlockSpec((_F, _F), lambda i: (0, 0)),
        ],
        out_specs=[
            pl.BlockSpec((blk, _F), lambda i: (i, 0)),
            pl.BlockSpec((blk, _F), lambda i: (i, 0)),
        ],
        out_shape=[
            jax.ShapeDtypeStruct((_NPAD, _F), jnp.bfloat16),
            jax.ShapeDtypeStruct((_NPAD, _F), jnp.float32),
        ],
    )(s, ng, Wn, b.reshape(1, -1), Wp, bp.reshape(1, -1), Ws)


def _mlp_body(le_ref, cmd_ref, Wc_ref, bc_ref, W3_ref, b3_ref, W4_ref, b4_ref,
              W5_ref, b5_ref, W6_ref, b6_ref, out_ref):
    enc = jnp.dot(cmd_ref[...], Wc_ref[...], preferred_element_type=jnp.float32) + bc_ref[...]
    prod = le_ref[...] * enc
    o = jax.nn.sigmoid(jnp.dot(prod, W3_ref[...], preferred_element_type=jnp.float32) + b3_ref[...])
    o = jax.nn.sigmoid(jnp.dot(o, W4_ref[...], preferred_element_type=jnp.float32) + b4_ref[...])
    o = jax.nn.sigmoid(jnp.dot(o, W5_ref[...], preferred_element_type=jnp.float32) + b5_ref[...])
    out_ref[...] = jax.nn.sigmoid(jnp.dot(o, W6_ref[...], preferred_element_type=jnp.float32) + b6_ref[...])


def _mlp(le, cmd, Wc, bc, W3, b3, W4, b4, W5, b5, W6, b6):
    return pl.pallas_call(
        _mlp_body,
        out_shape=jax.ShapeDtypeStruct((_LPAD, 1), jnp.float32),
    )(le, cmd.reshape(1, -1), Wc, bc.reshape(1, -1), W3, b3.reshape(1, -1),
      W4, b4.reshape(1, -1), W5, b5.reshape(1, -1), W6, b6.reshape(1, -1))


def kernel(node_inputs, edge_index, leaves, command, Wp1, bp1, Ws1, Wn1, b1,
           Wp2, bp2, Ws2, Wn2, b2, Wc, bc, W3, b3, W4, b4, W5, b5, W6, b6):
    N = node_inputs.shape[0]
    E = edge_index.shape[1]
    L = leaves.shape[0]

    x = jnp.pad(node_inputs, ((0, _NPAD - N), (0, 0)))
    src = edge_index[0]
    dst = edge_index[1]
    ep = -(-E // _CH) * _CH
    if (ep // _CH) % 2:
        ep += _CH
    if ep != E:
        src = jnp.pad(src, (0, ep - E))
        dst = jnp.pad(dst, (0, ep - E), constant_values=-1)

    def _pack(a):
        return lax.bitcast_convert_type(a.reshape(_NPAD, _FW, 2), jnp.int32)

    def _unpack(a):
        return lax.bitcast_convert_type(a, jnp.bfloat16).reshape(_NPAD, _F)

    hp1, s1 = _pre(x, Wp1, bp1, Ws1)
    ng1, sidx, sdl, counts = _segmax_scan(_pack(hp1), src, dst)
    hp2, s2 = _post_pre(s1, _unpack(ng1), Wn1, b1, Wp2, bp2, Ws2)
    ng2 = _segmax_replay(_pack(hp2), sidx, sdl, counts)
    h = _post(s2, _unpack(ng2), Wn2, b2)

    lv = jnp.pad(leaves, (0, _LPAD - L))
    le = _leafgather(h, lv)
    out = _mlp(le, command, Wc, bc, W3, b3, W4, b4, W5, b5, W6, b6)
    return out[:L]


# speedup vs baseline: 1.0296x; 1.0026x over previous
import functools

import jax
import jax.numpy as jnp
from jax import lax
from jax.experimental import pallas as pl
from jax.experimental.pallas import tpu as pltpu
from jax.experimental.pallas import tpu_sc as plsc

_NC, _NS = 2, 16
_NW = _NC * _NS
_PN = 320
_NPAD = _NW * _PN
_F = 128
_CH = 4032
_GH = 128
_G = 3 * _GH
_FLUSH_AT = _G - 32
_LPAD = 2048


_FW = _F // 2


def _zero_acc(acc):
    def zero_body(i, _):
        for j in range(_FW // 16):
            acc[i, pl.ds(16 * j, 16)] = jnp.zeros((16,), jnp.int32)
        return 0
    lax.fori_loop(0, _PN + 8, zero_body, 0)


def _merge_accs(*accs):
    acc0 = accs[0]

    def merge_body(i, _):
        for j in range(_FW // 16):
            a = plsc.bitcast(acc0[i, pl.ds(16 * j, 16)], jnp.bfloat16)
            for other in accs[1:]:
                b = plsc.bitcast(other[i, pl.ds(16 * j, 16)], jnp.bfloat16)
                a = jnp.maximum(a, b)
            acc0[i, pl.ds(16 * j, 16)] = plsc.bitcast(a, jnp.int32)
        return 0
    lax.fori_loop(0, _PN, merge_body, 0)


def _gather_and_accumulate(hp, idxb, dlb, rows, acc0, acc1, semg):
    cs = [pltpu.async_copy(hp.at[idxb.at[pl.ds(k * _GH, _GH)]], rows.at[k], semg)
          for k in range(_G // _GH)]
    for c in cs:
        c.wait()
    _accumulate_rows(dlb, rows, (acc0, acc1))


def _accumulate_rows(dlb, rows, accs):
    npar = len(accs)
    for b in range(_G // _GH):
        def grp_body(g, _, b=b):
            dvec = dlb[pl.ds(b * _GH + g * 16, 16)]
            for l in range(16):
                d = dvec[l]
                e = g * 16 + l
                acc = accs[l % npar]
                for j in range(_FW // 16):
                    a = plsc.bitcast(acc[d, pl.ds(16 * j, 16)], jnp.bfloat16)
                    r = plsc.bitcast(rows[b, e, pl.ds(16 * j, 16)], jnp.bfloat16)
                    acc[d, pl.ds(16 * j, 16)] = plsc.bitcast(
                        jnp.maximum(a, r), jnp.int32)
            return 0
        lax.fori_loop(0, _GH // 16, grp_body, 0)


def _segmax_scan_body(sb, hp, srcp, dstp, out, sidx, sdl, counts,
                      acc, acc1, srcb0, srcb1, dstb0, dstb1, idxb, dlb, rows,
                      stgi, stgd, cntv, semld, semg, semst):
    srcb = (srcb0, srcb1)
    dstb = (dstb0, dstb1)
    wid = lax.axis_index("s") * _NC + lax.axis_index("c")
    lo = wid * _PN
    tb = wid * (sb * _G)
    nchunks = srcp.shape[0] // _CH

    _zero_acc(acc)
    _zero_acc(acc1)
    for g in range(_G // 16 + 1):
        idxb[pl.ds(g * 16, 16)] = jnp.zeros((16,), jnp.int32)
        dlb[pl.ds(g * 16, 16)] = jnp.full((16,), _PN, jnp.int32)

    def flush(nb, nblk):
        del nb
        @pl.when(nblk > 0)
        def _():
            pltpu.make_async_copy(srcp.at[pl.ds(0, _G)], stgi, semst).wait()
            pltpu.make_async_copy(srcp.at[pl.ds(0, _G)], stgd, semst).wait()
        for g in range(_G // 16):
            stgi[pl.ds(g * 16, 16)] = idxb[pl.ds(g * 16, 16)]
            stgd[pl.ds(g * 16, 16)] = dlb[pl.ds(g * 16, 16)]
        off = tb + nblk * _G
        pltpu.async_copy(stgi, sidx.at[pl.ds(off, _G)], semst)
        pltpu.async_copy(stgd, sdl.at[pl.ds(off, _G)], semst)
        _gather_and_accumulate(hp, idxb, dlb, rows, acc, acc1, semg)

    def issue_chunk(ci, b):
        off = ci * _CH
        pltpu.async_copy(srcp.at[pl.ds(off, _CH)], srcb[b], semld.at[b])
        pltpu.async_copy(dstp.at[pl.ds(off, _CH)], dstb[b], semld.at[b])

    def wait_chunk(b):
        pltpu.make_async_copy(srcp.at[pl.ds(0, _CH)], srcb[b], semld.at[b]).wait()
        pltpu.make_async_copy(dstp.at[pl.ds(0, _CH)], dstb[b], semld.at[b]).wait()

    issue_chunk(0, 0)

    def outer_body(k, carry):
        for b in range(2):
            ci = 2 * k + b

            @pl.when(ci + 1 < nchunks)
            def _():
                issue_chunk(ci + 1, b ^ 1)
            wait_chunk(b)

            def pair_body(gp, carry):
                nb, nblk = carry
                for u in range(2):
                    base = gp * 32 + u * 16
                    dv = dstb[b][pl.ds(base, 16)]
                    sv = srcb[b][pl.ds(base, 16)]
                    dl = dv - lo
                    own = (dl >= 0) & (dl < _PN)
                    keys = jnp.where(own, dl, jnp.int32(_PN))
                    ks, vs = plsc.sort_key_val(keys, sv)
                    dlb[pl.ds(nb, 16)] = ks
                    idxb[pl.ds(nb, 16)] = vs
                    nb = nb + plsc.all_reduce_population_count(own)[0]

                def do_flush(args):
                    n, nk = args
                    flush(n, nk)
                    return jnp.int32(0), nk + 1
                return lax.cond(nb >= _FLUSH_AT, do_flush, lambda a: a, (nb, nblk))

            carry = lax.fori_loop(0, _CH // 32, pair_body, carry, unroll=2)
        return carry

    nb, nblk = lax.fori_loop(0, nchunks // 2, outer_body,
                             (jnp.int32(0), jnp.int32(0)))
    flush(nb, nblk)
    cntv[pl.ds(0, 16)] = jnp.full((16,), nblk + 1, jnp.int32)
    pltpu.sync_copy(cntv, counts.at[pl.ds(wid * 16, 16)])
    pltpu.make_async_copy(srcp.at[pl.ds(0, _G)], stgi, semst).wait()
    pltpu.make_async_copy(srcp.at[pl.ds(0, _G)], stgd, semst).wait()
    _merge_accs(acc, acc1)
    pltpu.sync_copy(acc.at[pl.ds(0, _PN)], out.at[pl.ds(lo, _PN)])


def _segmax_replay_body(sb, hp, sidx, sdl, counts, out,
                        acc, acc1, acc2, acc3, idxb, dlb, rows, cntv, semg):
    accs = (acc, acc1, acc2, acc3)
    wid = lax.axis_index("s") * _NC + lax.axis_index("c")
    lo = wid * _PN
    tb = wid * (sb * _G)

    for a in accs:
        _zero_acc(a)
    pltpu.sync_copy(counts.at[pl.ds(wid * 16, 16)], cntv)
    cnt = cntv[pl.ds(0, 16)][0]

    def blk_body(bi, _):
        off = tb + bi * _G
        pltpu.sync_copy(sidx.at[pl.ds(off, _G + 16)], idxb)
        pltpu.sync_copy(sdl.at[pl.ds(off, _G + 16)], dlb)
        cs = [pltpu.async_copy(hp.at[idxb.at[pl.ds(k * _GH, _GH)]],
                               rows.at[k], semg)
              for k in range(_G // _GH)]
        for c in cs:
            c.wait()
        _accumulate_rows(dlb, rows, accs)
        return 0

    lax.fori_loop(0, cnt, blk_body, 0)
    _merge_accs(*accs)
    pltpu.sync_copy(acc.at[pl.ds(0, _PN)], out.at[pl.ds(lo, _PN)])


def _sc_mesh():
    return plsc.VectorSubcoreMesh(core_axis_name="c", subcore_axis_name="s",
                                  num_cores=_NC, num_subcores=_NS)


def _segmax_scan(hp, src, dst):
    sb = src.shape[0] // _FLUSH_AT + 2
    sz = _NW * sb * _G + 16
    return pl.kernel(
        functools.partial(_segmax_scan_body, sb),
        out_type=[
            jax.ShapeDtypeStruct((_NPAD, _FW), jnp.int32),
            jax.ShapeDtypeStruct((sz,), jnp.int32),
            jax.ShapeDtypeStruct((sz,), jnp.int32),
            jax.ShapeDtypeStruct((_NW * 16,), jnp.int32),
        ],
        mesh=_sc_mesh(),
        scratch_types=[
            pltpu.VMEM((_PN + 8, _FW), jnp.int32),
            pltpu.VMEM((_PN + 8, _FW), jnp.int32),
            pltpu.VMEM((_CH,), jnp.int32),
            pltpu.VMEM((_CH,), jnp.int32),
            pltpu.VMEM((_CH,), jnp.int32),
            pltpu.VMEM((_CH,), jnp.int32),
            pltpu.VMEM((_G + 16,), jnp.int32),
            pltpu.VMEM((_G + 16,), jnp.int32),
            pltpu.VMEM((_G // _GH, _GH, _FW), jnp.int32),
            pltpu.VMEM((_G,), jnp.int32),
            pltpu.VMEM((_G,), jnp.int32),
            pltpu.VMEM((16,), jnp.int32),
            pltpu.SemaphoreType.DMA((2,)),
            pltpu.SemaphoreType.DMA,
            pltpu.SemaphoreType.DMA,
        ],
        compiler_params=pltpu.CompilerParams(needs_layout_passes=False, use_tc_tiling_on_sc=False),
        name="sc_segmax",
    )(hp, src, dst)


def _segmax_replay(hp, sidx, sdl, counts):
    sb = (sidx.shape[0] - 16) // (_NW * _G)
    return pl.kernel(
        functools.partial(_segmax_replay_body, sb),
        out_type=jax.ShapeDtypeStruct((_NPAD, _FW), jnp.int32),
        mesh=_sc_mesh(),
        scratch_types=[
            pltpu.VMEM((_PN + 8, _FW), jnp.int32),
            pltpu.VMEM((_PN + 8, _FW), jnp.int32),
            pltpu.VMEM((_PN + 8, _FW), jnp.int32),
            pltpu.VMEM((_PN + 8, _FW), jnp.int32),
            pltpu.VMEM((_G + 16,), jnp.int32),
            pltpu.VMEM((_G + 16,), jnp.int32),
            pltpu.VMEM((_G // _GH, _GH, _FW), jnp.int32),
            pltpu.VMEM((16,), jnp.int32),
            pltpu.SemaphoreType.DMA,
        ],
        compiler_params=pltpu.CompilerParams(needs_layout_passes=False, use_tc_tiling_on_sc=False),
        name="sc_segmax_replay",
    )(hp, sidx, sdl, counts)


def _leafgather_body(h, idx, out, idxv, rowsv, sem):
    wid = lax.axis_index("s") * _NC + lax.axis_index("c")
    per = _LPAD // _NW
    base = wid * per
    pltpu.sync_copy(idx.at[pl.ds(base, per)], idxv)
    pltpu.async_copy(h.at[idxv], rowsv, sem).wait()
    pltpu.sync_copy(rowsv, out.at[pl.ds(base, per)])


def _leafgather(h, idx):
    mesh = plsc.VectorSubcoreMesh(core_axis_name="c", subcore_axis_name="s",
                                  num_cores=_NC, num_subcores=_NS)
    per = _LPAD // _NW
    return pl.kernel(
        _leafgather_body,
        out_type=jax.ShapeDtypeStruct((_LPAD, _F), jnp.float32),
        mesh=mesh,
        scratch_types=[
            pltpu.VMEM((per,), jnp.int32),
            pltpu.VMEM((per, _F), jnp.float32),
            pltpu.SemaphoreType.DMA,
        ],
        name="sc_leafgather",
    )(h, idx)


def _pre_body(x_ref, Wp_ref, bp_ref, Ws_ref, hp_ref, s_ref):
    x = x_ref[...]
    hp_ref[...] = jnp.maximum(
        jnp.dot(x, Wp_ref[...], preferred_element_type=jnp.float32) + bp_ref[...],
        0.0).astype(jnp.bfloat16)
    s_ref[...] = jnp.dot(x, Ws_ref[...], preferred_element_type=jnp.float32)


def _pre(x, Wp, bp, Ws):
    grid = 8
    blk = _NPAD // grid
    return pl.pallas_call(
        _pre_body,
        grid=(grid,),
        in_specs=[
            pl.BlockSpec((blk, _F), lambda i: (i, 0)),
            pl.BlockSpec((_F, _F), lambda i: (0, 0)),
            pl.BlockSpec((1, _F), lambda i: (0, 0)),
            pl.BlockSpec((_F, _F), lambda i: (0, 0)),
        ],
        out_specs=[
            pl.BlockSpec((blk, _F), lambda i: (i, 0)),
            pl.BlockSpec((blk, _F), lambda i: (i, 0)),
        ],
        out_shape=[
            jax.ShapeDtypeStruct((_NPAD, _F), jnp.bfloat16),
            jax.ShapeDtypeStruct((_NPAD, _F), jnp.float32),
        ],
    )(x, Wp, bp.reshape(1, -1), Ws)


def _post_body(s_ref, ng_ref, Wn_ref, b_ref, h_ref):
    ng = ng_ref[...].astype(jnp.float32)
    h_ref[...] = jax.nn.sigmoid(
        s_ref[...]
        + jnp.dot(ng, Wn_ref[...], preferred_element_type=jnp.float32)
        + b_ref[...])


def _post(s, ng, Wn, b):
    grid = 8
    blk = _NPAD // grid
    return pl.pallas_call(
        _post_body,
        grid=(grid,),
        in_specs=[
            pl.BlockSpec((blk, _F), lambda i: (i, 0)),
            pl.BlockSpec((blk, _F), lambda i: (i, 0)),
            pl.BlockSpec((_F, _F), lambda i: (0, 0)),
            pl.BlockSpec((1, _F), lambda i: (0, 0)),
        ],
        out_specs=pl.BlockSpec((blk, _F), lambda i: (i, 0)),
        out_shape=jax.ShapeDtypeStruct((_NPAD, _F), jnp.float32),
    )(s, ng, Wn, b.reshape(1, -1))


def _post_pre_body(s_ref, ng_ref, Wn_ref, b_ref, Wp_ref, bp_ref, Ws_ref,
                   hp_ref, s2_ref):
    ng = ng_ref[...].astype(jnp.float32)
    h = jax.nn.sigmoid(
        s_ref[...]
        + jnp.dot(ng, Wn_ref[...], preferred_element_type=jnp.float32)
        + b_ref[...])
    hp_ref[...] = jnp.maximum(
        jnp.dot(h, Wp_ref[...], preferred_element_type=jnp.float32) + bp_ref[...],
        0.0).astype(jnp.bfloat16)
    s2_ref[...] = jnp.dot(h, Ws_ref[...], preferred_element_type=jnp.float32)


def _post_pre(s, ng, Wn, b, Wp, bp, Ws):
    grid = 8
    blk = _NPAD // grid
    return pl.pallas_call(
        _post_pre_body,
        grid=(grid,),
        in_specs=[
            pl.BlockSpec((blk, _F), lambda i: (i, 0)),
            pl.BlockSpec((blk, _F), lambda i: (i, 0)),
            pl.BlockSpec((_F, _F), lambda i: (0, 0)),
            pl.BlockSpec((1, _F), lambda i: (0, 0)),
            pl.BlockSpec((_F, _F), lambda i: (0, 0)),
            pl.BlockSpec((1, _F), lambda i: (0, 0)),
            pl.BlockSpec((_F, _F), lambda i: (0, 0)),
        ],
        out_specs=[
            pl.BlockSpec((blk, _F), lambda i: (i, 0)),
            pl.BlockSpec((blk, _F), lambda i: (i, 0)),
        ],
        out_shape=[
            jax.ShapeDtypeStruct((_NPAD, _F), jnp.bfloat16),
            jax.ShapeDtypeStruct((_NPAD, _F), jnp.float32),
        ],
    )(s, ng, Wn, b.reshape(1, -1), Wp, bp.reshape(1, -1), Ws)


def _mlp_body(le_ref, cmd_ref, Wc_ref, bc_ref, W3_ref, b3_ref, W4_ref, b4_ref,
              W5_ref, b5_ref, W6_ref, b6_ref, out_ref):
    enc = jnp.dot(cmd_ref[...], Wc_ref[...], preferred_element_type=jnp.float32) + bc_ref[...]
    prod = le_ref[...] * enc
    o = jax.nn.sigmoid(jnp.dot(prod, W3_ref[...], preferred_element_type=jnp.float32) + b3_ref[...])
    o = jax.nn.sigmoid(jnp.dot(o, W4_ref[...], preferred_element_type=jnp.float32) + b4_ref[...])
    o = jax.nn.sigmoid(jnp.dot(o, W5_ref[...], preferred_element_type=jnp.float32) + b5_ref[...])
    out_ref[...] = jax.nn.sigmoid(jnp.dot(o, W6_ref[...], preferred_element_type=jnp.float32) + b6_ref[...])


def _mlp(le, cmd, Wc, bc, W3, b3, W4, b4, W5, b5, W6, b6):
    return pl.pallas_call(
        _mlp_body,
        out_shape=jax.ShapeDtypeStruct((_LPAD, 1), jnp.float32),
    )(le, cmd.reshape(1, -1), Wc, bc.reshape(1, -1), W3, b3.reshape(1, -1),
      W4, b4.reshape(1, -1), W5, b5.reshape(1, -1), W6, b6.reshape(1, -1))


def kernel(node_inputs, edge_index, leaves, command, Wp1, bp1, Ws1, Wn1, b1,
           Wp2, bp2, Ws2, Wn2, b2, Wc, bc, W3, b3, W4, b4, W5, b5, W6, b6):
    N = node_inputs.shape[0]
    E = edge_index.shape[1]
    L = leaves.shape[0]

    x = jnp.pad(node_inputs, ((0, _NPAD - N), (0, 0)))
    src = edge_index[0]
    dst = edge_index[1]
    ep = -(-E // _CH) * _CH
    if (ep // _CH) % 2:
        ep += _CH
    if ep != E:
        src = jnp.pad(src, (0, ep - E))
        dst = jnp.pad(dst, (0, ep - E), constant_values=-1)

    def _pack(a):
        return lax.bitcast_convert_type(a.reshape(_NPAD, _FW, 2), jnp.int32)

    def _unpack(a):
        return lax.bitcast_convert_type(a, jnp.bfloat16).reshape(_NPAD, _F)

    hp1, s1 = _pre(x, Wp1, bp1, Ws1)
    ng1, sidx, sdl, counts = _segmax_scan(_pack(hp1), src, dst)
    hp2, s2 = _post_pre(s1, _unpack(ng1), Wn1, b1, Wp2, bp2, Ws2)
    ng2 = _segmax_replay(_pack(hp2), sidx, sdl, counts)
    h = _post(s2, _unpack(ng2), Wn2, b2)

    lv = jnp.pad(leaves, (0, _LPAD - L))
    le = _leafgather(h, lv)
    out = _mlp(le, command, Wc, bc, W3, b3, W4, b4, W5, b5, W6, b6)
    return out[:L]
